# baseline (device time: 9836706 ns/iter reference)
import jax
import jax.numpy as jnp
from jax import lax
from jax.experimental import pallas as pl
from jax.experimental.pallas import tpu as pltpu


def kernel(x, dest):
    T, D = x.shape
    NBITS = T.bit_length()
    SUB = D // 128

    combined = dest.astype(jnp.int32) * T + jnp.arange(T, dtype=jnp.int32)
    order = jnp.sort(combined) & (T - 1)
    c0 = (T - jnp.sum(dest)).astype(jnp.int32).reshape((1,))
    col_starts = jnp.arange(SUB, dtype=jnp.int32) * 128
    gidx = jnp.stack(
        jnp.broadcast_arrays(order[:, None], col_starts[None, :]), axis=-1
    )
    x_sorted = lax.gather(
        x,
        gidx,
        dimension_numbers=lax.GatherDimensionNumbers(
            offset_dims=(2,), collapsed_slice_dims=(0,), start_index_map=(0, 1)
        ),
        slice_sizes=(1, 128),
        mode=lax.GatherScatterMode.PROMISE_IN_BOUNDS,
    )

    def body(x_ref, c0_ref, out_ref, stage_ref, send_sems, recv_sems):
        my_x = lax.axis_index("x")
        my_y = lax.axis_index("y")
        my_z = lax.axis_index("z")
        peer = (1 - my_x, my_y, my_z)

        barrier = pltpu.get_barrier_semaphore()
        pl.semaphore_signal(
            barrier, inc=1, device_id=peer, device_id_type=pl.DeviceIdType.MESH
        )
        pl.semaphore_wait(barrier, 1)

        c0v = c0_ref[0]
        zero = jnp.int32(0)
        is0 = my_x == 0
        own_off = jnp.where(is0, zero, c0v)
        own_len = jnp.where(is0, c0v, T - c0v)
        comm_len = T - own_len
        send_src = jnp.where(is0, c0v, zero)
        remote_dst = jnp.where(is0, zero, T - comm_len)

        soff = send_src
        roff = remote_dst
        for b in reversed(range(NBITS)):
            L = 1 << b
            take = comm_len & L

            @pl.when(take != 0)
            def _(soff=soff, roff=roff, L=L, b=b):
                stage_ref[pl.ds(soff, L), :, :] = x_ref[
                    pl.ds(soff, L), :, :
                ].astype(jnp.bfloat16)
                pltpu.make_async_remote_copy(
                    src_ref=stage_ref.at[pl.ds(soff, L)],
                    dst_ref=out_ref.at[pl.ds(roff, L)],
                    send_sem=send_sems.at[b],
                    recv_sem=recv_sems.at[b],
                    device_id=peer,
                    device_id_type=pl.DeviceIdType.MESH,
                ).start()

            soff = soff + take
            roff = roff + take

        off = own_off
        for b in reversed(range(NBITS)):
            L = 1 << b
            take = own_len & L

            @pl.when(take != 0)
            def _(off=off, L=L):
                out_ref[pl.ds(off, L), :, :] = x_ref[
                    pl.ds(off, L), :, :
                ].astype(jnp.bfloat16)

            off = off + take

        for b in range(NBITS):
            L = 1 << b
            take = comm_len & L

            @pl.when(take != 0)
            def _(L=L, b=b):
                rdma = pltpu.make_async_remote_copy(
                    src_ref=stage_ref.at[pl.ds(0, L)],
                    dst_ref=out_ref.at[pl.ds(0, L)],
                    send_sem=send_sems.at[b],
                    recv_sem=recv_sems.at[b],
                    device_id=peer,
                    device_id_type=pl.DeviceIdType.MESH,
                )
                rdma.wait_send()
                rdma.wait_recv()

    out = pl.pallas_call(
        body,
        out_shape=jax.ShapeDtypeStruct((T, SUB, 128), jnp.bfloat16),
        in_specs=[
            pl.BlockSpec(memory_space=pltpu.VMEM),
            pl.BlockSpec(memory_space=pltpu.SMEM),
        ],
        out_specs=pl.BlockSpec(memory_space=pltpu.VMEM),
        scratch_shapes=[
            pltpu.VMEM((T, SUB, 128), jnp.bfloat16),
            pltpu.SemaphoreType.DMA((NBITS,)),
            pltpu.SemaphoreType.DMA((NBITS,)),
        ],
        compiler_params=pltpu.CompilerParams(collective_id=0),
    )(x_sorted, c0)
    return out.reshape(T, D)


# device time: 149666 ns/iter; 65.7244x vs baseline; 65.7244x over previous
import jax
import jax.numpy as jnp
from jax import lax
from jax.experimental import pallas as pl
from jax.experimental.pallas import tpu as pltpu


def kernel(x, dest):
    T, D = x.shape
    NBITS = T.bit_length()
    SUB = D // 128

    combined = dest.astype(jnp.int32) * T + jnp.arange(T, dtype=jnp.int32)
    order = jnp.sort(combined) & (T - 1)
    c0 = (T - jnp.sum(dest)).astype(jnp.int32).reshape((1,))
    x_sorted = jnp.take(x, order, axis=0).reshape(T, SUB, 128)

    def body(x_ref, c0_ref, out_ref, stage_ref, send_sems, recv_sems):
        my_x = lax.axis_index("x")
        my_y = lax.axis_index("y")
        my_z = lax.axis_index("z")
        peer = (1 - my_x, my_y, my_z)

        barrier = pltpu.get_barrier_semaphore()
        pl.semaphore_signal(
            barrier, inc=1, device_id=peer, device_id_type=pl.DeviceIdType.MESH
        )
        pl.semaphore_wait(barrier, 1)

        c0v = c0_ref[0]
        zero = jnp.int32(0)
        is0 = my_x == 0
        own_off = jnp.where(is0, zero, c0v)
        own_len = jnp.where(is0, c0v, T - c0v)
        comm_len = T - own_len
        send_src = jnp.where(is0, c0v, zero)
        remote_dst = jnp.where(is0, zero, T - comm_len)

        soff = send_src
        roff = remote_dst
        for b in reversed(range(NBITS)):
            L = 1 << b
            take = comm_len & L

            @pl.when(take != 0)
            def _(soff=soff, roff=roff, L=L, b=b):
                stage_ref[pl.ds(soff, L), :, :] = x_ref[
                    pl.ds(soff, L), :, :
                ].astype(jnp.bfloat16)
                pltpu.make_async_remote_copy(
                    src_ref=stage_ref.at[pl.ds(soff, L)],
                    dst_ref=out_ref.at[pl.ds(roff, L)],
                    send_sem=send_sems.at[b],
                    recv_sem=recv_sems.at[b],
                    device_id=peer,
                    device_id_type=pl.DeviceIdType.MESH,
                ).start()

            soff = soff + take
            roff = roff + take

        off = own_off
        for b in reversed(range(NBITS)):
            L = 1 << b
            take = own_len & L

            @pl.when(take != 0)
            def _(off=off, L=L):
                out_ref[pl.ds(off, L), :, :] = x_ref[
                    pl.ds(off, L), :, :
                ].astype(jnp.bfloat16)

            off = off + take

        for b in range(NBITS):
            L = 1 << b
            take = comm_len & L

            @pl.when(take != 0)
            def _(L=L, b=b):
                rdma = pltpu.make_async_remote_copy(
                    src_ref=stage_ref.at[pl.ds(0, L)],
                    dst_ref=out_ref.at[pl.ds(0, L)],
                    send_sem=send_sems.at[b],
                    recv_sem=recv_sems.at[b],
                    device_id=peer,
                    device_id_type=pl.DeviceIdType.MESH,
                )
                rdma.wait_send()
                rdma.wait_recv()

    out = pl.pallas_call(
        body,
        out_shape=jax.ShapeDtypeStruct((T, SUB, 128), jnp.bfloat16),
        in_specs=[
            pl.BlockSpec(memory_space=pltpu.VMEM),
            pl.BlockSpec(memory_space=pltpu.SMEM),
        ],
        out_specs=pl.BlockSpec(memory_space=pltpu.VMEM),
        scratch_shapes=[
            pltpu.VMEM((T, SUB, 128), jnp.bfloat16),
            pltpu.SemaphoreType.DMA((NBITS,)),
            pltpu.SemaphoreType.DMA((NBITS,)),
        ],
        compiler_params=pltpu.CompilerParams(collective_id=0),
    )(x_sorted, c0)
    return out.reshape(T, D)


# device time: 51667 ns/iter; 190.3866x vs baseline; 2.8967x over previous
import jax
import jax.numpy as jnp
from jax import lax
from jax.experimental import pallas as pl
from jax.experimental.pallas import tpu as pltpu


def kernel(x, dest):
    T, D = x.shape
    NBITS = T.bit_length()
    SUB = D // 128

    combined = dest.astype(jnp.int32) * T + jnp.arange(T, dtype=jnp.int32)
    order = jnp.sort(combined) & (T - 1)
    c0 = (T - jnp.sum(dest)).astype(jnp.int32).reshape((1,))
    x_sorted = jnp.take(x, order, axis=0).astype(jnp.bfloat16).reshape(T, SUB, 128)

    def body(x_ref, c0_ref, out_ref, send_sems, recv_sems, copy_sems):
        my_x = lax.axis_index("x")
        my_y = lax.axis_index("y")
        my_z = lax.axis_index("z")
        peer = (1 - my_x, my_y, my_z)

        barrier = pltpu.get_barrier_semaphore()
        pl.semaphore_signal(
            barrier, inc=1, device_id=peer, device_id_type=pl.DeviceIdType.MESH
        )
        pl.semaphore_wait(barrier, 1)

        c0v = c0_ref[0]
        zero = jnp.int32(0)
        is0 = my_x == 0
        own_off = jnp.where(is0, zero, c0v)
        own_len = jnp.where(is0, c0v, T - c0v)
        comm_len = T - own_len
        send_src = jnp.where(is0, c0v, zero)
        remote_dst = jnp.where(is0, zero, T - comm_len)

        soff = send_src
        roff = remote_dst
        for b in reversed(range(NBITS)):
            L = 1 << b
            take = comm_len & L

            @pl.when(take != 0)
            def _(soff=soff, roff=roff, L=L, b=b):
                pltpu.make_async_remote_copy(
                    src_ref=x_ref.at[pl.ds(soff, L)],
                    dst_ref=out_ref.at[pl.ds(roff, L)],
                    send_sem=send_sems.at[b],
                    recv_sem=recv_sems.at[b],
                    device_id=peer,
                    device_id_type=pl.DeviceIdType.MESH,
                ).start()

            soff = soff + take
            roff = roff + take

        off = own_off
        for b in reversed(range(NBITS)):
            L = 1 << b
            take = own_len & L

            @pl.when(take != 0)
            def _(off=off, L=L, b=b):
                pltpu.make_async_copy(
                    x_ref.at[pl.ds(off, L)],
                    out_ref.at[pl.ds(off, L)],
                    copy_sems.at[b],
                ).start()

            off = off + take

        for b in range(NBITS):
            L = 1 << b

            @pl.when((own_len & L) != 0)
            def _(L=L, b=b):
                pltpu.make_async_copy(
                    x_ref.at[pl.ds(0, L)],
                    out_ref.at[pl.ds(0, L)],
                    copy_sems.at[b],
                ).wait()

            @pl.when((comm_len & L) != 0)
            def _(L=L, b=b):
                rdma = pltpu.make_async_remote_copy(
                    src_ref=x_ref.at[pl.ds(0, L)],
                    dst_ref=out_ref.at[pl.ds(0, L)],
                    send_sem=send_sems.at[b],
                    recv_sem=recv_sems.at[b],
                    device_id=peer,
                    device_id_type=pl.DeviceIdType.MESH,
                )
                rdma.wait_send()
                rdma.wait_recv()

    out = pl.pallas_call(
        body,
        out_shape=jax.ShapeDtypeStruct((T, SUB, 128), jnp.bfloat16),
        in_specs=[
            pl.BlockSpec(memory_space=pl.ANY),
            pl.BlockSpec(memory_space=pltpu.SMEM),
        ],
        out_specs=pl.BlockSpec(memory_space=pl.ANY),
        scratch_shapes=[
            pltpu.SemaphoreType.DMA((NBITS,)),
            pltpu.SemaphoreType.DMA((NBITS,)),
            pltpu.SemaphoreType.DMA((NBITS,)),
        ],
        compiler_params=pltpu.CompilerParams(collective_id=0),
    )(x_sorted, c0)
    return out.reshape(T, D)
